# baseline (speedup 1.0000x reference)
import functools

import jax
import jax.numpy as jnp
from jax import lax
from jax.experimental import pallas as pl
from jax.experimental.pallas import tpu as pltpu
from jax.experimental.pallas import tpu_sc as plsc

HIDDEN = 64
BATCH = 16384
NGROUPS = 62500

_NC = 2
_NS = 16
_NW = _NC * _NS
_BPW = BATCH // _NW
_CHUNK = 32
_NCHUNK = _BPW // _CHUNK


def _sc_gather(table3, ids3):
    mesh = plsc.VectorSubcoreMesh(core_axis_name="c", subcore_axis_name="s")

    @functools.partial(
        pl.kernel,
        out_type=jax.ShapeDtypeStruct((BATCH, 8, 2 * HIDDEN), jnp.float32),
        mesh=mesh,
        scratch_types=[
            pltpu.VMEM((_NCHUNK, _CHUNK), jnp.int32),
            pltpu.VMEM((2, _CHUNK, 8, 2 * HIDDEN), jnp.float32),
            pltpu.SemaphoreType.DMA,
            pltpu.SemaphoreType.DMA,
        ],
    )
    def k(table_hbm, ids_hbm, out_hbm, idx_v, tiles_v, gsem, osem):
        wid = lax.axis_index("s") * _NC + lax.axis_index("c")
        base = wid * _BPW
        pltpu.sync_copy(ids_hbm.at[wid], idx_v)
        for j in range(_NCHUNK):
            for t in range(_CHUNK // 16):
                sl = pl.ds(t * 16, 16)
                idx_v[j, sl] = lax.shift_right_logical(idx_v[j, sl], 4)

        outs = [None, None]
        for j in range(_NCHUNK):
            b = j % 2
            if outs[b] is not None:
                outs[b].wait()
                outs[b] = None
            pltpu.async_copy(
                table_hbm.at[idx_v.at[j]], tiles_v.at[b], gsem).wait()
            outs[b] = pltpu.async_copy(
                tiles_v.at[b], out_hbm.at[pl.ds(base + j * _CHUNK, _CHUNK)],
                osem)
        for b in range(2):
            if outs[b] is not None:
                outs[b].wait()

    return k(table3, ids3)


def _tc_body(x_ref, pe3_ref, r_ref, h_ref, a_ref, p_ref, ae_ref, pt_ref,
             w_ref, b_ref, g_ref, bt_ref, o_ref):
    r = r_ref[...]
    sel = jnp.where(r == 0, pe3_ref[:, 0, :], 0.0)
    for k in range(1, 8):
        sel += jnp.where(r == k, pe3_ref[:, k, :], 0.0)
    hlf = h_ref[...]
    h = x_ref[...] + jnp.where(hlf == 0, sel[:, :HIDDEN], sel[:, HIDDEN:])
    a = a_ref[...]
    for k in range(3):
        h += jnp.where(a == k, ae_ref[k, :][None, :], 0.0)
    p = p_ref[...]
    for k in range(10):
        h += jnp.where(p == k, pt_ref[k, :][None, :], 0.0)
    hw = lax.dot_general(h, w_ref[...], (((1,), (1,)), ((), ())),
                         preferred_element_type=jnp.float32) + b_ref[...]
    mean = jnp.mean(hw, axis=1, keepdims=True)
    cen = hw - mean
    var = jnp.mean(cen * cen, axis=1, keepdims=True)
    o_ref[...] = cen * lax.rsqrt(var + 1e-5) * g_ref[...] + bt_ref[...]


def _tc_fused(x, pe3, r2, h2, a2, p2, action_emb, pos_emb, W, b2, g2, bt2,
              blk=2048):
    grid = BATCH // blk
    return pl.pallas_call(
        _tc_body,
        grid=(grid,),
        in_specs=[
            pl.BlockSpec((blk, HIDDEN), lambda i: (i, 0)),
            pl.BlockSpec((blk, 8, 2 * HIDDEN), lambda i: (i, 0, 0)),
            pl.BlockSpec((blk, 1), lambda i: (i, 0)),
            pl.BlockSpec((blk, 1), lambda i: (i, 0)),
            pl.BlockSpec((blk, 1), lambda i: (i, 0)),
            pl.BlockSpec((blk, 1), lambda i: (i, 0)),
            pl.BlockSpec((3, HIDDEN), lambda i: (0, 0)),
            pl.BlockSpec((10, HIDDEN), lambda i: (0, 0)),
            pl.BlockSpec((HIDDEN, HIDDEN), lambda i: (0, 0)),
            pl.BlockSpec((1, HIDDEN), lambda i: (0, 0)),
            pl.BlockSpec((1, HIDDEN), lambda i: (0, 0)),
            pl.BlockSpec((1, HIDDEN), lambda i: (0, 0)),
        ],
        out_specs=pl.BlockSpec((blk, HIDDEN), lambda i: (i, 0)),
        out_shape=jax.ShapeDtypeStruct((BATCH, HIDDEN), jnp.float32),
    )(x, pe3, r2, h2, a2, p2, action_emb, pos_emb, W, b2, g2, bt2)


def kernel(x, player_ids, actions, positions, player_emb, action_emb,
           pos_emb, W, b, gamma, beta):
    ids = player_ids.astype(jnp.int32)
    ids3 = ids.reshape(_NW, _NCHUNK, _CHUNK)
    table3 = player_emb.reshape(NGROUPS, 8, 2 * HIDDEN)
    pe3 = _sc_gather(table3, ids3)
    r2 = ((ids % 16) // 2).reshape(BATCH, 1)
    h2 = (ids % 2).reshape(BATCH, 1)
    a2 = actions.astype(jnp.int32).reshape(BATCH, 1)
    p2 = positions.astype(jnp.int32).reshape(BATCH, 1)
    out = _tc_fused(x, pe3, r2, h2, a2, p2, action_emb, pos_emb, W,
                    b.reshape(1, HIDDEN), gamma.reshape(1, HIDDEN),
                    beta.reshape(1, HIDDEN))
    return out.reshape(BATCH, 1, HIDDEN)


# speedup vs baseline: 2.3690x; 2.3690x over previous
import functools

import jax
import jax.numpy as jnp
from jax import lax
from jax.experimental import pallas as pl
from jax.experimental.pallas import tpu as pltpu
from jax.experimental.pallas import tpu_sc as plsc

HIDDEN = 64
BATCH = 16384
NPLAYERS = 1000000
NCOLS = 7812
TAIL = NCOLS * 128
SINK = BATCH

_CPW = 245
_C = 4
_NCH = 62
_LANES = _C * 128


def _sc_stream_gather(tbl_t, ids_hbm):
    mesh = plsc.VectorSubcoreMesh(core_axis_name="c", subcore_axis_name="s")

    @functools.partial(
        pl.kernel,
        out_type=jax.ShapeDtypeStruct((BATCH + 64, 128), jnp.float32),
        mesh=mesh,
        compiler_params=pltpu.CompilerParams(needs_layout_passes=False),
        scratch_types=[
            pltpu.VMEM((1024,), jnp.int32),
            pltpu.VMEM((BATCH + 16,), jnp.int32),
            pltpu.VMEM((BATCH + 16,), jnp.int32),
            pltpu.VMEM((2, 64, _LANES), jnp.float32),
            pltpu.VMEM((128, 128), jnp.float32),
            pltpu.VMEM((2, 64), jnp.int32),
            pltpu.VMEM((32,), jnp.int32),
            pltpu.VMEM((32,), jnp.int32),
            pltpu.SMEM((8,), jnp.int32),
            pltpu.SemaphoreType.DMA,
        ],
    )
    def k(tbl_hbm, ids_in, out_hbm, idbuf, lids, lb, stage, orow, obidx,
          pend_p, pend_b, sc, sem):
        wid = lax.axis_index("s") * 2 + lax.axis_index("c")
        lo = jnp.minimum(wid * _CPW, NCOLS)
        hi = jnp.minimum(lo + _CPW, NCOLS)
        lo_p = lo * 128
        hi_p = hi * 128
        i16 = lax.broadcasted_iota(jnp.int32, (16,), 0)

        for q in range(8):
            obidx[q // 4, pl.ds((q % 4) * 16, 16)] = \
                SINK + i16 + (q % 4) * 16

        n = jnp.int32(0)
        for t in range(16):
            pltpu.sync_copy(ids_in.at[pl.ds(t * 1024, 1024)], idbuf)

            def bodyA(v, n, t=t):
                p = idbuf[pl.ds(v * 16, 16)]
                m = (p >= lo_p) & (p < hi_p)
                b = i16 + (t * 1024 + v * 16)
                mi = jnp.where(m, 1, 0)
                cs = plsc.cumsum(mi)
                ranks = n + cs - mi
                plsc.store_scatter(lids, [ranks], p, mask=m)
                plsc.store_scatter(lb, [ranks], b, mask=m)
                return n + jnp.max(cs)

            n = lax.fori_loop(0, 64, bodyA, n)
        sc[0] = 0
        sc[1] = 0
        sc[2] = 0
        sc[3] = n

        def extract(bufv, s, O, cnt):
            hp = pend_p[pl.ds(0, 16)]
            hb = pend_b[pl.ds(0, 16)]
            hm = i16 < cnt
            colidx = (lax.shift_right_logical(hp, 7) - s) * 128 + (hp & 127)
            rpos = (O + i16) & 127
            plsc.store_scatter(
                obidx, [lax.shift_right_logical(rpos, 6), rpos & 63], hb,
                mask=hm)
            bufvec = i16 * 0 + bufv

            def fbody(q, c):
                for u in range(4):
                    fv = i16 * 0 + (q * 4 + u)
                    vals = plsc.load_gather(stage, [bufvec, fv, colidx],
                                            mask=hm)
                    plsc.store_scatter(orow, [rpos, fv], vals, mask=hm)
                return c

            lax.fori_loop(0, 16, fbody, jnp.int32(0))

        def flush(F):
            slot = lax.shift_right_logical(F, 6) & 1
            pltpu.async_copy(
                orow.at[pl.ds(slot * 64, 64)],
                out_hbm.at[obidx.at[slot]], sem).wait()
            for q in range(4):
                obidx[slot, pl.ds(q * 16, 16)] = SINK + i16 + q * 16

        def chunk_body(kk, c):
            bufv = kk & 1
            s = jnp.maximum(lo, jnp.minimum(lo + kk * _C, hi - _C))
            pltpu.sync_copy(
                tbl_hbm.at[pl.ds(0, 64), pl.ds(s * 128, _LANES)],
                stage.at[bufv])
            nn = sc[3]
            ns = (nn + 15) // 16

            def bodyB(g, c2):
                O = sc[0]
                F = sc[1]
                ph = sc[2]
                p = lids[pl.ds(g * 16, 16)]
                b = lb[pl.ds(g * 16, 16)]
                m = (i16 < (nn - g * 16)) & (p >= s * 128) & \
                    (p < (s + _C) * 128)
                mi = jnp.where(m, 1, 0)
                cs = plsc.cumsum(mi)
                ranks = ph + cs - mi
                plsc.store_scatter(pend_p, [ranks], p, mask=m)
                plsc.store_scatter(pend_b, [ranks], b, mask=m)
                ph2 = ph + jnp.max(cs)
                sc[2] = ph2

                @pl.when(ph2 >= 16)
                def _():
                    extract(bufv, s, O, jnp.int32(16))
                    rp = pend_p[pl.ds(16, 16)]
                    rb = pend_b[pl.ds(16, 16)]
                    pend_p[pl.ds(0, 16)] = rp
                    pend_b[pl.ds(0, 16)] = rb
                    sc[0] = O + 16
                    sc[2] = ph2 - 16

                    @pl.when(O + 16 - F >= 64)
                    def _():
                        flush(F)
                        sc[1] = F + 64

                return c2

            lax.fori_loop(0, ns, bodyB, jnp.int32(0))
            O = sc[0]
            F = sc[1]
            ph = sc[2]
            extract(bufv, s, O, ph)
            sc[0] = O + ph
            sc[2] = 0

            @pl.when(O + ph - F >= 64)
            def _():
                flush(F)
                sc[1] = F + 64

            return c

        lax.fori_loop(0, _NCH, chunk_body, jnp.int32(0))

        @pl.when(sc[0] - sc[1] > 0)
        def _():
            flush(sc[1])

    return k(tbl_t, ids_hbm)


def _tc_body(x_ref, pe_ref, pid_ref, tail_ref, a_ref, p_ref, ae_ref,
             pt_ref, w_ref, b_ref, g_ref, bt_ref, o_ref):
    pid = pid_ref[...]
    is_tail = pid >= TAIL
    lane = lax.broadcasted_iota(jnp.int32, (1, HIDDEN), 1)
    oh = jnp.where(is_tail & ((pid - TAIL) == lane), 1.0, 0.0)
    pe_tail = lax.dot_general(oh, tail_ref[...], (((1,), (0,)), ((), ())),
                              preferred_element_type=jnp.float32)
    pe = jnp.where(is_tail, pe_tail, pe_ref[:, :HIDDEN])
    h = x_ref[...] + pe
    a = a_ref[...]
    for k in range(3):
        h += jnp.where(a == k, ae_ref[k, :][None, :], 0.0)
    p = p_ref[...]
    for k in range(10):
        h += jnp.where(p == k, pt_ref[k, :][None, :], 0.0)
    hw = lax.dot_general(h, w_ref[...], (((1,), (1,)), ((), ())),
                         preferred_element_type=jnp.float32) + b_ref[...]
    mean = jnp.mean(hw, axis=1, keepdims=True)
    cen = hw - mean
    var = jnp.mean(cen * cen, axis=1, keepdims=True)
    o_ref[...] = cen * lax.rsqrt(var + 1e-5) * g_ref[...] + bt_ref[...]


def _tc_fused(x, pe, pid2, tail, a2, p2, action_emb, pos_emb, W, b2, g2,
              bt2, blk=2048):
    grid = BATCH // blk
    return pl.pallas_call(
        _tc_body,
        grid=(grid,),
        in_specs=[
            pl.BlockSpec((blk, HIDDEN), lambda i: (i, 0)),
            pl.BlockSpec((blk, 2 * HIDDEN), lambda i: (i, 0)),
            pl.BlockSpec((blk, 1), lambda i: (i, 0)),
            pl.BlockSpec((HIDDEN, HIDDEN), lambda i: (0, 0)),
            pl.BlockSpec((blk, 1), lambda i: (i, 0)),
            pl.BlockSpec((blk, 1), lambda i: (i, 0)),
            pl.BlockSpec((3, HIDDEN), lambda i: (0, 0)),
            pl.BlockSpec((10, HIDDEN), lambda i: (0, 0)),
            pl.BlockSpec((HIDDEN, HIDDEN), lambda i: (0, 0)),
            pl.BlockSpec((1, HIDDEN), lambda i: (0, 0)),
            pl.BlockSpec((1, HIDDEN), lambda i: (0, 0)),
            pl.BlockSpec((1, HIDDEN), lambda i: (0, 0)),
        ],
        out_specs=pl.BlockSpec((blk, HIDDEN), lambda i: (i, 0)),
        out_shape=jax.ShapeDtypeStruct((BATCH, HIDDEN), jnp.float32),
    )(x, pe, pid2, tail, a2, p2, action_emb, pos_emb, W, b2, g2, bt2)


def kernel(x, player_ids, actions, positions, player_emb, action_emb,
           pos_emb, W, b, gamma, beta):
    ids = player_ids.astype(jnp.int32)
    peS = _sc_stream_gather(player_emb.T, ids)[:BATCH]
    pid2 = ids.reshape(BATCH, 1)
    tail = lax.slice(player_emb, (TAIL, 0), (NPLAYERS, HIDDEN))
    a2 = actions.astype(jnp.int32).reshape(BATCH, 1)
    p2 = positions.astype(jnp.int32).reshape(BATCH, 1)
    out = _tc_fused(x, peS, pid2, tail, a2, p2, action_emb, pos_emb, W,
                    b.reshape(1, HIDDEN), gamma.reshape(1, HIDDEN),
                    beta.reshape(1, HIDDEN))
    return out.reshape(BATCH, 1, HIDDEN)


# speedup vs baseline: 3.4730x; 1.4660x over previous
import functools

import jax
import jax.numpy as jnp
from jax import lax
from jax.experimental import pallas as pl
from jax.experimental.pallas import tpu as pltpu
from jax.experimental.pallas import tpu_sc as plsc

HIDDEN = 64
BATCH = 16384
NPLAYERS = 1000000
NCOLS = 7812
TAIL = NCOLS * 128
SINK = BATCH

_CPW = 245
_C = 4
_NCH = 62
_LANES = _C * 128


def _sc_stream_gather(tbl_t, ids_hbm):
    mesh = plsc.VectorSubcoreMesh(core_axis_name="c", subcore_axis_name="s")

    @functools.partial(
        pl.kernel,
        out_type=jax.ShapeDtypeStruct((BATCH + 64, 128), jnp.float32),
        mesh=mesh,
        compiler_params=pltpu.CompilerParams(needs_layout_passes=False),
        scratch_types=[
            pltpu.VMEM((1024,), jnp.int32),
            pltpu.VMEM((BATCH + 16,), jnp.int32),
            pltpu.VMEM((BATCH + 16,), jnp.int32),
            pltpu.VMEM((2, 64, _LANES), jnp.float32),
            pltpu.VMEM((128, 128), jnp.float32),
            pltpu.VMEM((2, 64), jnp.int32),
            pltpu.VMEM((32,), jnp.int32),
            pltpu.VMEM((32,), jnp.int32),
            pltpu.SMEM((8,), jnp.int32),
            pltpu.SemaphoreType.DMA,
            pltpu.SemaphoreType.DMA,
            pltpu.SemaphoreType.DMA,
        ],
    )
    def k(tbl_hbm, ids_in, out_hbm, idbuf, lids, lb, stage, orow, obidx,
          pend_p, pend_b, sc, sem, semA, semB):
        wid = lax.axis_index("s") * 2 + lax.axis_index("c")
        lo = jnp.minimum(wid * _CPW, NCOLS)
        hi = jnp.minimum(lo + _CPW, NCOLS)
        lo_p = lo * 128
        hi_p = hi * 128
        i16 = lax.broadcasted_iota(jnp.int32, (16,), 0)

        for q in range(8):
            obidx[q // 4, pl.ds((q % 4) * 16, 16)] = \
                SINK + i16 + (q % 4) * 16

        n = jnp.int32(0)
        for t in range(16):
            pltpu.sync_copy(ids_in.at[pl.ds(t * 1024, 1024)], idbuf)

            def bodyA(v, n, t=t):
                p = idbuf[pl.ds(v * 16, 16)]
                m = (p >= lo_p) & (p < hi_p)
                b = i16 + (t * 1024 + v * 16)
                mi = jnp.where(m, 1, 0)
                cs = plsc.cumsum(mi)
                ranks = n + cs - mi
                plsc.store_scatter(lids, [ranks], p, mask=m)
                plsc.store_scatter(lb, [ranks], b, mask=m)
                return n + jnp.sum(mi)

            n = lax.fori_loop(0, 64, bodyA, n)
        sc[0] = 0
        sc[1] = 0
        sc[2] = 0
        sc[3] = n

        def extract(bufv, s, O, cnt):
            hp = pend_p[pl.ds(0, 16)]
            hb = pend_b[pl.ds(0, 16)]
            hm = i16 < cnt
            colidx = (lax.shift_right_logical(hp, 7) - s) * 128 + (hp & 127)
            rpos = (O + i16) & 127
            plsc.store_scatter(
                obidx, [lax.shift_right_logical(rpos, 6), rpos & 63], hb,
                mask=hm)
            bufvec = i16 * 0 + bufv

            def fbody(q, c):
                for u in range(4):
                    fv = i16 * 0 + (q * 4 + u)
                    vals = plsc.load_gather(stage, [bufvec, fv, colidx],
                                            mask=hm)
                    plsc.store_scatter(orow, [rpos, fv], vals, mask=hm)
                return c

            lax.fori_loop(0, 16, fbody, jnp.int32(0))

        def flush(F):
            slot = lax.shift_right_logical(F, 6) & 1
            pltpu.async_copy(
                orow.at[pl.ds(slot * 64, 64)],
                out_hbm.at[obidx.at[slot]], sem).wait()
            for q in range(4):
                obidx[slot, pl.ds(q * 16, 16)] = SINK + i16 + q * 16

        def stage_src(kk):
            s = jnp.maximum(lo, jnp.minimum(lo + kk * _C, hi - _C))
            return tbl_hbm.at[pl.ds(0, 64), pl.ds(s * 128, _LANES)]

        def chunk_body(kk, c):
            bufv = kk & 1
            s = jnp.maximum(lo, jnp.minimum(lo + kk * _C, hi - _C))
            even = (kk & 1) == 0
            more = kk < (_NCH - 1)

            @pl.when(even & more)
            def _():
                pltpu.async_copy(stage_src(kk + 1), stage.at[1], semB)

            @pl.when((~even) & more)
            def _():
                pltpu.async_copy(stage_src(kk + 1), stage.at[0], semA)

            @pl.when(even)
            def _():
                pltpu.make_async_copy(stage_src(kk), stage.at[0],
                                      semA).wait()

            @pl.when(~even)
            def _():
                pltpu.make_async_copy(stage_src(kk), stage.at[1],
                                      semB).wait()

            nn = sc[3]
            ns = (nn + 15) // 16

            def bodyB(g, c2):
                O = sc[0]
                F = sc[1]
                ph = sc[2]
                p = lids[pl.ds(g * 16, 16)]
                b = lb[pl.ds(g * 16, 16)]
                m = (i16 < (nn - g * 16)) & (p >= s * 128) & \
                    (p < (s + _C) * 128)
                mi = jnp.where(m, 1, 0)
                cs = plsc.cumsum(mi)
                ranks = ph + cs - mi
                plsc.store_scatter(pend_p, [ranks], p, mask=m)
                plsc.store_scatter(pend_b, [ranks], b, mask=m)
                ph2 = ph + jnp.sum(mi)
                sc[2] = ph2

                @pl.when(ph2 >= 16)
                def _():
                    extract(bufv, s, O, jnp.int32(16))
                    rp = pend_p[pl.ds(16, 16)]
                    rb = pend_b[pl.ds(16, 16)]
                    pend_p[pl.ds(0, 16)] = rp
                    pend_b[pl.ds(0, 16)] = rb
                    sc[0] = O + 16
                    sc[2] = ph2 - 16

                    @pl.when(O + 16 - F >= 64)
                    def _():
                        flush(F)
                        sc[1] = F + 64

                return c2

            lax.fori_loop(0, ns, bodyB, jnp.int32(0))
            O = sc[0]
            F = sc[1]
            ph = sc[2]
            extract(bufv, s, O, ph)
            sc[0] = O + ph
            sc[2] = 0

            @pl.when(O + ph - F >= 64)
            def _():
                flush(F)
                sc[1] = F + 64

            return c

        pltpu.async_copy(stage_src(jnp.int32(0)), stage.at[0], semA)
        lax.fori_loop(0, _NCH, chunk_body, jnp.int32(0))

        @pl.when(sc[0] - sc[1] > 0)
        def _():
            flush(sc[1])

    return k(tbl_t, ids_hbm)


def _tc_body(x_ref, pe_ref, pid_ref, tail_ref, a_ref, p_ref, ae_ref,
             pt_ref, w_ref, b_ref, g_ref, bt_ref, o_ref):
    pid = pid_ref[...]
    is_tail = pid >= TAIL
    lane = lax.broadcasted_iota(jnp.int32, (1, HIDDEN), 1)
    oh = jnp.where(is_tail & ((pid - TAIL) == lane), 1.0, 0.0)
    pe_tail = lax.dot_general(oh, tail_ref[...], (((1,), (0,)), ((), ())),
                              preferred_element_type=jnp.float32)
    pe = jnp.where(is_tail, pe_tail, pe_ref[:, :HIDDEN])
    h = x_ref[...] + pe
    a = a_ref[...]
    for k in range(3):
        h += jnp.where(a == k, ae_ref[k, :][None, :], 0.0)
    p = p_ref[...]
    for k in range(10):
        h += jnp.where(p == k, pt_ref[k, :][None, :], 0.0)
    hw = lax.dot_general(h, w_ref[...], (((1,), (1,)), ((), ())),
                         preferred_element_type=jnp.float32) + b_ref[...]
    mean = jnp.mean(hw, axis=1, keepdims=True)
    cen = hw - mean
    var = jnp.mean(cen * cen, axis=1, keepdims=True)
    o_ref[...] = cen * lax.rsqrt(var + 1e-5) * g_ref[...] + bt_ref[...]


def _tc_fused(x, pe, pid2, tail, a2, p2, action_emb, pos_emb, W, b2, g2,
              bt2, blk=2048):
    grid = BATCH // blk
    return pl.pallas_call(
        _tc_body,
        grid=(grid,),
        in_specs=[
            pl.BlockSpec((blk, HIDDEN), lambda i: (i, 0)),
            pl.BlockSpec((blk, 2 * HIDDEN), lambda i: (i, 0)),
            pl.BlockSpec((blk, 1), lambda i: (i, 0)),
            pl.BlockSpec((HIDDEN, HIDDEN), lambda i: (0, 0)),
            pl.BlockSpec((blk, 1), lambda i: (i, 0)),
            pl.BlockSpec((blk, 1), lambda i: (i, 0)),
            pl.BlockSpec((3, HIDDEN), lambda i: (0, 0)),
            pl.BlockSpec((10, HIDDEN), lambda i: (0, 0)),
            pl.BlockSpec((HIDDEN, HIDDEN), lambda i: (0, 0)),
            pl.BlockSpec((1, HIDDEN), lambda i: (0, 0)),
            pl.BlockSpec((1, HIDDEN), lambda i: (0, 0)),
            pl.BlockSpec((1, HIDDEN), lambda i: (0, 0)),
        ],
        out_specs=pl.BlockSpec((blk, HIDDEN), lambda i: (i, 0)),
        out_shape=jax.ShapeDtypeStruct((BATCH, HIDDEN), jnp.float32),
    )(x, pe, pid2, tail, a2, p2, action_emb, pos_emb, W, b2, g2, bt2)


def kernel(x, player_ids, actions, positions, player_emb, action_emb,
           pos_emb, W, b, gamma, beta):
    ids = player_ids.astype(jnp.int32)
    peS = _sc_stream_gather(player_emb.T, ids)[:BATCH]
    pid2 = ids.reshape(BATCH, 1)
    tail = lax.slice(player_emb, (TAIL, 0), (NPLAYERS, HIDDEN))
    a2 = actions.astype(jnp.int32).reshape(BATCH, 1)
    p2 = positions.astype(jnp.int32).reshape(BATCH, 1)
    out = _tc_fused(x, peS, pid2, tail, a2, p2, action_emb, pos_emb, W,
                    b.reshape(1, HIDDEN), gamma.reshape(1, HIDDEN),
                    beta.reshape(1, HIDDEN))
    return out.reshape(BATCH, 1, HIDDEN)
